# baseline (speedup 1.0000x reference)
import functools
import jax
import jax.numpy as jnp
from jax import lax
from jax.experimental import pallas as pl
from jax.experimental.pallas import tpu as pltpu
from jax.experimental.pallas import tpu_sc as plsc

N = 10000
E = 160000
D = 256
G = 64
NC = 2
NS = 16
L = 16

ROWS_PER_SUB = N // NS
ROWS_A = 624
TAIL_A = N - NS * ROWS_A
BLK = 400
NBLK = N // BLK

_mesh = plsc.VectorSubcoreMesh(core_axis_name="c", subcore_axis_name="s")


def _iota16():
    return lax.broadcasted_iota(jnp.int32, (L,), 0)


def _elem(ref, i):
    return ref[pl.ds(i, L)][0]


DEG_E = 5000
DEG_CH = 5
DEG_CW = 1000


@functools.partial(
    pl.kernel,
    out_type=jax.ShapeDtypeStruct((NC * N,), jnp.float32),
    mesh=_mesh,
    compiler_params=pltpu.CompilerParams(needs_layout_passes=False),
    scratch_types=[
        pltpu.VMEM((DEG_E,), jnp.int32),
        pltpu.VMEM((DEG_E,), jnp.int32),
        pltpu.VMEM((DEG_CW,), jnp.float32),
        pltpu.VMEM((DEG_CW,), jnp.float32),
        pltpu.VMEM((ROWS_A,), jnp.float32),
        pltpu.VMEM_SHARED((N,), jnp.float32),
        pltpu.SemaphoreType.DMA,
        pltpu.SemaphoreType.DMA,
    ],
)
def _deg_kernel(keepf, srcv, dstv, rdeg_out, src_v, dst_v, kv, kv_b, zb, acc,
                dsem0, dsem1):
    c = lax.axis_index("c")
    s = lax.axis_index("s")
    wid = s * NC + c

    def zfill(j, carry):
        zb[pl.ds(j * L, L)] = jnp.zeros((L,), jnp.float32)
        return carry

    lax.fori_loop(0, ROWS_A // L, zfill, 0)
    pltpu.sync_copy(zb, acc.at[pl.ds(s * ROWS_A, ROWS_A)])

    @pl.when(s == NS - 1)
    def _():
        pltpu.sync_copy(zb.at[pl.ds(0, TAIL_A)],
                        acc.at[pl.ds(NS * ROWS_A, TAIL_A)])

    plsc.subcore_barrier()
    pltpu.sync_copy(srcv.at[wid], src_v)
    pltpu.sync_copy(dstv.at[wid], dst_v)

    kbufs = (kv, kv_b)
    ksems = (dsem0, dsem1)
    kcps = [None, None]
    kcps[0] = pltpu.async_copy(
        keepf.at[src_v.at[pl.ds(0, DEG_CW)]], kv, dsem0)
    for j in range(DEG_CH):
        nxt = j + 1
        if nxt < DEG_CH:
            kcps[nxt % 2] = pltpu.async_copy(
                keepf.at[src_v.at[pl.ds(nxt * DEG_CW, DEG_CW)]],
                kbufs[nxt % 2], ksems[nxt % 2])
        kcps[j % 2].wait()
        pltpu.sync_copy(kbufs[j % 2],
                        acc.at[dst_v.at[pl.ds(j * DEG_CW, DEG_CW)]], add=True)
    plsc.subcore_barrier()
    pltpu.sync_copy(acc.at[pl.ds(s * ROWS_A, ROWS_A)], zb)
    pltpu.sync_copy(zb, rdeg_out.at[pl.ds(c * N + s * ROWS_A, ROWS_A)])

    @pl.when(s == NS - 1)
    def _():
        pltpu.sync_copy(acc.at[pl.ds(NS * ROWS_A, TAIL_A)],
                        kv.at[pl.ds(0, TAIL_A)])
        pltpu.sync_copy(kv.at[pl.ds(0, TAIL_A)],
                        rdeg_out.at[pl.ds(c * N + NS * ROWS_A, TAIL_A)])


DH = 128
CONV_CH = 80
CONV_CW = 125


WB_CH = 48
CONV_SH = CONV_CH // 2


@functools.partial(
    pl.kernel,
    out_type=jax.ShapeDtypeStruct((NC * N, DH), jnp.float32),
    mesh=_mesh,
    compiler_params=pltpu.CompilerParams(needs_layout_passes=False),
    scratch_types=[
        pltpu.VMEM((CONV_SH, CONV_CW), jnp.int32),
        pltpu.VMEM((CONV_SH, CONV_CW), jnp.int32),
        pltpu.VMEM((CONV_CW, DH), jnp.float32),
        pltpu.VMEM((CONV_CW, DH), jnp.float32),
        pltpu.VMEM((WB_CH, DH), jnp.float32),
        pltpu.VMEM_SHARED((N, DH), jnp.float32),
        pltpu.SemaphoreType.DMA,
        pltpu.SemaphoreType.DMA,
    ],
)
def _conv_kernel(y2, srcv, dstv, agg_out, src_v, dst_v, rows_a, rows_b,
                 zb, acc, sem0, sem1):
    c = lax.axis_index("c")
    s = lax.axis_index("s")

    def zfill(r, carry):
        for j in range(DH // L):
            zb[r, pl.ds(j * L, L)] = jnp.zeros((L,), jnp.float32)
        return carry

    lax.fori_loop(0, WB_CH, zfill, 0)
    for t in range(ROWS_A // WB_CH):
        pltpu.sync_copy(zb, acc.at[pl.ds(s * ROWS_A + t * WB_CH, WB_CH)])

    @pl.when(s == NS - 1)
    def _():
        pltpu.sync_copy(zb.at[pl.ds(0, TAIL_A)],
                        acc.at[pl.ds(NS * ROWS_A, TAIL_A)])

    plsc.subcore_barrier()
    y_half = y2.at[pl.ds(c * N, N)]

    bufs = (rows_a, rows_b)
    sems = (sem0, sem1)
    cps = [None, None]
    for half in range(2):
        pltpu.sync_copy(srcv.at[s].at[pl.ds(half * CONV_SH, CONV_SH)], src_v)
        pltpu.sync_copy(dstv.at[s].at[pl.ds(half * CONV_SH, CONV_SH)], dst_v)
        cps[0] = pltpu.async_copy(y_half.at[src_v.at[0]], rows_a, sem0)
        for j in range(CONV_SH):
            nxt = j + 1
            if nxt < CONV_SH:
                cps[nxt % 2] = pltpu.async_copy(
                    y_half.at[src_v.at[nxt]], bufs[nxt % 2], sems[nxt % 2])
            cps[j % 2].wait()
            pltpu.sync_copy(bufs[j % 2], acc.at[dst_v.at[j]], add=True)
    plsc.subcore_barrier()
    for t in range(ROWS_A // WB_CH):
        pltpu.sync_copy(acc.at[pl.ds(s * ROWS_A + t * WB_CH, WB_CH)], zb)
        pltpu.sync_copy(
            zb, agg_out.at[pl.ds(c * N + s * ROWS_A + t * WB_CH, WB_CH)])

    @pl.when(s == NS - 1)
    def _():
        pltpu.sync_copy(acc.at[pl.ds(NS * ROWS_A, TAIL_A)],
                        zb.at[pl.ds(0, TAIL_A)])
        pltpu.sync_copy(zb.at[pl.ds(0, TAIL_A)],
                        agg_out.at[pl.ds(c * N + NS * ROWS_A, TAIL_A)])


def _starts_body(b_ref, o_ref):
    i = pl.program_id(0)

    @pl.when(i == 0)
    def _():
        o_ref[...] = jnp.zeros((1, 128), jnp.int32)

    g = lax.broadcasted_iota(jnp.int32, (1, 128), 1)
    cmp = (b_ref[...] < g).astype(jnp.int32)
    o_ref[...] += jnp.sum(cmp, axis=0, keepdims=True)


def _starts_call(batch):
    return pl.pallas_call(
        _starts_body,
        grid=(NBLK,),
        in_specs=[pl.BlockSpec((BLK, 1), lambda i: (i, 0))],
        out_specs=pl.BlockSpec((1, 128), lambda i: (0, 0)),
        out_shape=jax.ShapeDtypeStruct((1, 128), jnp.int32),
    )(batch.reshape(N, 1)).reshape(128)


XCH = 64
TKW = 2048


@functools.partial(
    pl.kernel,
    out_type=[
        jax.ShapeDtypeStruct((N + L,), jnp.float32),
        jax.ShapeDtypeStruct((G, D), jnp.float32),
        jax.ShapeDtypeStruct((G, D), jnp.float32),
        jax.ShapeDtypeStruct((G, L), jnp.float32),
    ],
    mesh=_mesh,
    compiler_params=pltpu.CompilerParams(needs_layout_passes=False),
    scratch_types=[
        pltpu.VMEM((128,), jnp.int32),
        pltpu.VMEM((N,), jnp.uint32),
        pltpu.VMEM((N,), jnp.float32),
        pltpu.VMEM((N + 128,), jnp.float32),
        pltpu.VMEM((128,), jnp.int32),
        pltpu.VMEM((XCH, D), jnp.float32),
        pltpu.VMEM((2, D), jnp.float32),
        pltpu.VMEM((2, D), jnp.float32),
        pltpu.VMEM((2, L), jnp.float32),
        pltpu.VMEM((L,), jnp.float32),
        pltpu.SemaphoreType.DMA,
    ],
)
def _topk_kernel(starts_hbm, ukey_hbm, keep_hbm, xnew_hbm,
                 keepn_out, mx_out, sm_out, kn_out,
                 starts_v, ukey_v, keep_v, kn_v, tg_v, xbuf, mx_l, sm_l,
                 knv_l, stg, sem):
    c = lax.axis_index("c")
    s = lax.axis_index("s")
    wid = s * NC + c
    pltpu.sync_copy(starts_hbm, starts_v)

    iota = _iota16()

    starts = [_elem(starts_v, 2 * wid), _elem(starts_v, 2 * wid + 1),
              _elem(starts_v, 2 * wid + 2)]

    base8 = jnp.minimum(lax.div(starts[0], 8) * 8, N - TKW)
    fits = (starts[2] - base8) <= TKW

    @pl.when(fits)
    def _():
        pltpu.sync_copy(ukey_hbm.at[pl.ds(base8, TKW)],
                        ukey_v.at[pl.ds(base8, TKW)])
        pltpu.sync_copy(keep_hbm.at[pl.ds(base8, TKW)],
                        keep_v.at[pl.ds(base8, TKW)])

    @pl.when(jnp.logical_not(fits))
    def _():
        pltpu.sync_copy(ukey_hbm, ukey_v)
        pltpu.sync_copy(keep_hbm, keep_v)

    for gi in range(2):
        ga = starts[gi]
        gb = starts[gi + 1]
        g = 2 * wid + gi
        ca = lax.div(ga, L)
        cb = lax.div(gb + L - 1, L)

        def masked_count(pred_fn):
            def body(jj, acc):
                base = jj * L
                iv = base + iota
                uk = ukey_v[pl.ds(base, L)]
                kf = keep_v[pl.ds(base, L)]
                m = (kf > 0.0) & (iv >= ga) & (iv < gb) & pred_fn(iv, uk)
                return acc + plsc.all_reduce_population_count(m)
            return lax.fori_loop(ca, cb, body, jnp.zeros((L,), jnp.int32))

        cnt = masked_count(lambda iv, uk: jnp.full((L,), True))
        k = lax.shift_right_logical(cnt + 1, 1)

        def bs_step(_, st):
            lo, hi, cgt = st
            mid = lo + lax.shift_right_logical(hi - lo, jnp.uint32(1))
            cm = masked_count(lambda iv, uk, midv=mid: uk > midv)
            ge = cm >= k
            lo2 = jnp.where(ge, mid, lo)
            hi2 = jnp.where(ge, hi, mid)
            cgt2 = jnp.where(ge, cgt, cm)
            return lo2, hi2, cgt2

        ulo = jnp.zeros((L,), jnp.uint32)
        uhi = jnp.full((L,), 0xFFFFFFFF, jnp.uint32)
        _, vthr, cgt = lax.fori_loop(
            0, 32, bs_step, (ulo, uhi, jnp.zeros((L,), jnp.int32)))
        needed = k - cgt

        def ti_step(_, lh):
            lo, hi = lh
            mid = lax.div(lo + hi, 2)
            cm = masked_count(
                lambda iv, uk, midv=mid: (uk == vthr) & (iv <= midv))
            ge = cm >= needed
            return jnp.where(ge, lo, mid + 1), jnp.where(ge, mid, hi)

        ilo = jnp.full((L,), ga - 1, jnp.int32)
        ihi = jnp.full((L,), gb - 1, jnp.int32)
        _, tie_i = lax.fori_loop(0, 14, ti_step, (ilo, ihi))

        kpos = k > 0
        def wr_chunk(jj, carry):
            base = jj * L
            iv = base + iota
            uk = ukey_v[pl.ds(base, L)]
            kf = keep_v[pl.ds(base, L)]
            ing = (iv >= ga) & (iv < gb)
            m = (kf > 0.0) & kpos & ((uk > vthr) | ((uk == vthr) & (iv <= tie_i)))
            vals = jnp.where(m, 1.0, 0.0).astype(jnp.float32)
            old = kn_v[pl.ds(base, L)]
            kn_v[pl.ds(base, L)] = jnp.where(ing, vals, old)
            return carry

        lax.fori_loop(ca, cb, wr_chunk, 0)

        for cg in range(D // L):
            mx_l[gi, pl.ds(cg * L, L)] = jnp.full((L,), -jnp.inf, jnp.float32)
            sm_l[gi, pl.ds(cg * L, L)] = jnp.zeros((L,), jnp.float32)

        def ro_chunk(jj, carry):
            base = jj * XCH
            pltpu.async_copy(xnew_hbm.at[pl.ds(base, XCH)], xbuf, sem).wait()

            def ro_row(r, c2):
                i = base + r
                sk = _elem(kn_v, i)
                ok = (sk > 0.0) & (i >= ga) & (i < gb)

                @pl.when(ok)
                def _():
                    for cg in range(D // L):
                        xv = xbuf[r, pl.ds(cg * L, L)]
                        mv = mx_l[gi, pl.ds(cg * L, L)]
                        sv = sm_l[gi, pl.ds(cg * L, L)]
                        mx_l[gi, pl.ds(cg * L, L)] = jnp.maximum(mv, xv)
                        sm_l[gi, pl.ds(cg * L, L)] = sv + xv
                return c2

            lax.fori_loop(0, XCH, ro_row, 0)
            return carry

        xa = lax.div(ga, XCH)
        xb = lax.div(gb + XCH - 1, XCH)
        lax.fori_loop(xa, xb, ro_chunk, 0)
        knv_l[gi] = jnp.where(kpos, k, 0).astype(jnp.float32)

    ga0 = starts[0]
    gb1 = starts[2]
    ca0 = lax.div(ga0, L)
    cb1 = lax.div(gb1 + L - 1, L)
    nbat = lax.div(cb1 - ca0 + 7, 8)

    def sc_bat(bat, carry):
        for t in range(8):
            jj = ca0 + bat * 8 + t
            iv = jj * L + iota
            ok = (jj < cb1) & (iv >= ga0) & (iv < gb1)
            tg_v[pl.ds(t * L, L)] = jnp.where(ok, iv, N + iota)
        pltpu.sync_copy(kn_v.at[pl.ds(ca0 * L + bat * 128, 128)],
                        keepn_out.at[tg_v])
        return carry

    lax.fori_loop(0, nbat, sc_bat, 0)

    pltpu.sync_copy(mx_l, mx_out.at[pl.ds(2 * wid, 2)])
    pltpu.sync_copy(sm_l, sm_out.at[pl.ds(2 * wid, 2)])
    pltpu.sync_copy(knv_l, kn_out.at[pl.ds(2 * wid, 2)])


def _mm_body(x_ref, w_ref, ra_ref, rb_ref, kf_ref, y_ref):
    xw = jnp.dot(x_ref[...], w_ref[...], preferred_element_type=jnp.float32)
    rdeg = ra_ref[...] + rb_ref[...]
    dis = kf_ref[...] * lax.rsqrt(rdeg + 1.0)
    y_ref[...] = dis * xw


def _mm_call(x, W, rdeg_part, keepf):
    ra = rdeg_part[:N].reshape(N, 1)
    rb = rdeg_part[N:].reshape(N, 1)
    kf = keepf.reshape(N, 1)
    return pl.pallas_call(
        _mm_body,
        grid=(2, NBLK),
        in_specs=[
            pl.BlockSpec((BLK, D), lambda h, i: (i, 0)),
            pl.BlockSpec((D, DH), lambda h, i: (0, h)),
            pl.BlockSpec((BLK, 1), lambda h, i: (i, 0)),
            pl.BlockSpec((BLK, 1), lambda h, i: (i, 0)),
            pl.BlockSpec((BLK, 1), lambda h, i: (i, 0)),
        ],
        out_specs=pl.BlockSpec((BLK, DH), lambda h, i: (h * NBLK + i, 0)),
        out_shape=jax.ShapeDtypeStruct((NC * N, DH), jnp.float32),
    )(x, W, ra, rb, kf)


def _post_body(a0_ref, a1_ref, y0_ref, y1_ref, ra_ref, rb_ref, kf_ref,
               b_ref, p_ref, xn_ref, uk_ref):
    agg = jnp.concatenate([a0_ref[...], a1_ref[...]], axis=1)
    y = jnp.concatenate([y0_ref[...], y1_ref[...]], axis=1)
    rdeg = ra_ref[...] + rb_ref[...]
    dis = kf_ref[...] * lax.rsqrt(rdeg + 1.0)
    h = jnp.maximum(dis * (agg + y) + b_ref[...], 0.0)
    p = p_ref[...]
    pn = jnp.sqrt(jnp.sum(p * p)) + 1e-16
    sc = jnp.tanh(jnp.dot(h, p.reshape(D, 1),
                          preferred_element_type=jnp.float32) / pn)
    xn_ref[...] = h * sc
    ub = lax.bitcast_convert_type(sc, jnp.uint32)
    uk_ref[...] = jnp.where(ub < jnp.uint32(0x80000000),
                            ub | jnp.uint32(0x80000000),
                            ~ub)


def _post_call(agg2, y2, rdeg_part, keepf, b, p):
    ra = rdeg_part[:N].reshape(N, 1)
    rb = rdeg_part[N:].reshape(N, 1)
    kf = keepf.reshape(N, 1)
    return pl.pallas_call(
        _post_body,
        grid=(NBLK,),
        in_specs=[
            pl.BlockSpec((BLK, DH), lambda i: (i, 0)),
            pl.BlockSpec((BLK, DH), lambda i: (NBLK + i, 0)),
            pl.BlockSpec((BLK, DH), lambda i: (i, 0)),
            pl.BlockSpec((BLK, DH), lambda i: (NBLK + i, 0)),
            pl.BlockSpec((BLK, 1), lambda i: (i, 0)),
            pl.BlockSpec((BLK, 1), lambda i: (i, 0)),
            pl.BlockSpec((BLK, 1), lambda i: (i, 0)),
            pl.BlockSpec((1, D), lambda i: (0, 0)),
            pl.BlockSpec((1, D), lambda i: (0, 0)),
        ],
        out_specs=[
            pl.BlockSpec((BLK, D), lambda i: (i, 0)),
            pl.BlockSpec((BLK, 1), lambda i: (i, 0)),
        ],
        out_shape=[
            jax.ShapeDtypeStruct((N, D), jnp.float32),
            jax.ShapeDtypeStruct((N, 1), jnp.uint32),
        ],
    )(agg2, agg2, y2, y2, ra, rb, kf, b.reshape(1, D), p.reshape(1, D))


def _combine_body(m0, s0, k0, m1, s1, k1, m2, s2, k2, o_ref):
    mx = m0[...] + m1[...] + m2[...]
    mean = (s0[...] / jnp.maximum(k0[...][:, :1], 1.0)
            + s1[...] / jnp.maximum(k1[...][:, :1], 1.0)
            + s2[...] / jnp.maximum(k2[...][:, :1], 1.0))
    o_ref[...] = jnp.concatenate([mx, mean], axis=1)


def _combine_call(parts):
    args = []
    for (mx, sm, kn) in parts:
        args += [mx, sm, kn]
    return pl.pallas_call(
        _combine_body,
        out_shape=jax.ShapeDtypeStruct((G, 2 * D), jnp.float32),
    )(*args)


def kernel(x, edge_index, batch, W0, b0, p0, W1, b1, p1, W2, b2, p2):
    src = edge_index[0]
    dst = edge_index[1]
    src_deg = src.reshape(NC * NS, DEG_E)
    dst_deg = dst.reshape(NC * NS, DEG_E)
    src_conv = src.reshape(NS, CONV_CH, CONV_CW)
    dst_conv = dst.reshape(NS, CONV_CH, CONV_CW)

    keepf = jnp.ones((N,), jnp.float32)
    starts = _starts_call(batch)
    parts = []
    for (W, b, p) in ((W0, b0, p0), (W1, b1, p1), (W2, b2, p2)):
        rdeg_part = _deg_kernel(keepf, src_deg, dst_deg)
        y2 = _mm_call(x, W, rdeg_part, keepf)
        agg2 = _conv_kernel(y2, src_conv, dst_conv)
        x, ukey = _post_call(agg2, y2, rdeg_part, keepf, b, p)
        keep_pad, mx, sm, kn = _topk_kernel(
            starts, ukey.reshape(N), keepf, x)
        keepf = keep_pad[:N]
        parts.append((mx, sm, kn))
    return _combine_call(parts)


# speedup vs baseline: 1.5544x; 1.5544x over previous
import functools
import jax
import jax.numpy as jnp
from jax import lax
from jax.experimental import pallas as pl
from jax.experimental.pallas import tpu as pltpu
from jax.experimental.pallas import tpu_sc as plsc

N = 10000
E = 160000
D = 256
G = 64
NC = 2
NS = 16
L = 16

ROWS_PER_SUB = N // NS
ROWS_A = 624
TAIL_A = N - NS * ROWS_A
BLK = 400
NBLK = N // BLK

_mesh = plsc.VectorSubcoreMesh(core_axis_name="c", subcore_axis_name="s")


def _iota16():
    return lax.broadcasted_iota(jnp.int32, (L,), 0)


def _elem(ref, i):
    return ref[pl.ds(i, L)][0]


DEG_E = 5000
DEG_CH = 5
DEG_CW = 1000


@functools.partial(
    pl.kernel,
    out_type=jax.ShapeDtypeStruct((NC * N,), jnp.float32),
    mesh=_mesh,
    compiler_params=pltpu.CompilerParams(needs_layout_passes=False),
    scratch_types=[
        pltpu.VMEM((DEG_E,), jnp.int32),
        pltpu.VMEM((DEG_E,), jnp.int32),
        pltpu.VMEM((DEG_CW,), jnp.float32),
        pltpu.VMEM((DEG_CW,), jnp.float32),
        pltpu.VMEM((ROWS_A,), jnp.float32),
        pltpu.VMEM_SHARED((N,), jnp.float32),
        pltpu.SemaphoreType.DMA,
        pltpu.SemaphoreType.DMA,
    ],
)
def _deg_kernel(keepf, srcv, dstv, rdeg_out, src_v, dst_v, kv, kv_b, zb, acc,
                dsem0, dsem1):
    c = lax.axis_index("c")
    s = lax.axis_index("s")
    wid = s * NC + c

    def zfill(j, carry):
        zb[pl.ds(j * L, L)] = jnp.zeros((L,), jnp.float32)
        return carry

    lax.fori_loop(0, ROWS_A // L, zfill, 0)
    pltpu.sync_copy(zb, acc.at[pl.ds(s * ROWS_A, ROWS_A)])

    @pl.when(s == NS - 1)
    def _():
        pltpu.sync_copy(zb.at[pl.ds(0, TAIL_A)],
                        acc.at[pl.ds(NS * ROWS_A, TAIL_A)])

    plsc.subcore_barrier()
    pltpu.sync_copy(srcv.at[wid], src_v)
    pltpu.sync_copy(dstv.at[wid], dst_v)

    kbufs = (kv, kv_b)
    ksems = (dsem0, dsem1)
    kcps = [None, None]
    kcps[0] = pltpu.async_copy(
        keepf.at[src_v.at[pl.ds(0, DEG_CW)]], kv, dsem0)
    for j in range(DEG_CH):
        nxt = j + 1
        if nxt < DEG_CH:
            kcps[nxt % 2] = pltpu.async_copy(
                keepf.at[src_v.at[pl.ds(nxt * DEG_CW, DEG_CW)]],
                kbufs[nxt % 2], ksems[nxt % 2])
        kcps[j % 2].wait()
        pltpu.sync_copy(kbufs[j % 2],
                        acc.at[dst_v.at[pl.ds(j * DEG_CW, DEG_CW)]], add=True)
    plsc.subcore_barrier()
    pltpu.sync_copy(acc.at[pl.ds(s * ROWS_A, ROWS_A)], zb)
    pltpu.sync_copy(zb, rdeg_out.at[pl.ds(c * N + s * ROWS_A, ROWS_A)])

    @pl.when(s == NS - 1)
    def _():
        pltpu.sync_copy(acc.at[pl.ds(NS * ROWS_A, TAIL_A)],
                        kv.at[pl.ds(0, TAIL_A)])
        pltpu.sync_copy(kv.at[pl.ds(0, TAIL_A)],
                        rdeg_out.at[pl.ds(c * N + NS * ROWS_A, TAIL_A)])


DH = 128
CONV_CH = 80
CONV_CW = 125


WB_CH = 48
CONV_SH = CONV_CH // 2


@functools.partial(
    pl.kernel,
    out_type=jax.ShapeDtypeStruct((NC * N, DH), jnp.float32),
    mesh=_mesh,
    compiler_params=pltpu.CompilerParams(needs_layout_passes=False),
    scratch_types=[
        pltpu.VMEM((CONV_SH, CONV_CW), jnp.int32),
        pltpu.VMEM((CONV_SH, CONV_CW), jnp.int32),
        pltpu.VMEM((CONV_CW, DH), jnp.float32),
        pltpu.VMEM((CONV_CW, DH), jnp.float32),
        pltpu.VMEM((WB_CH, DH), jnp.float32),
        pltpu.VMEM_SHARED((N, DH), jnp.float32),
        pltpu.SemaphoreType.DMA,
        pltpu.SemaphoreType.DMA,
    ],
)
def _conv_kernel(y2, srcv, dstv, agg_out, src_v, dst_v, rows_a, rows_b,
                 zb, acc, sem0, sem1):
    c = lax.axis_index("c")
    s = lax.axis_index("s")

    def zfill(r, carry):
        for j in range(DH // L):
            zb[r, pl.ds(j * L, L)] = jnp.zeros((L,), jnp.float32)
        return carry

    lax.fori_loop(0, WB_CH, zfill, 0)
    for t in range(ROWS_A // WB_CH):
        pltpu.sync_copy(zb, acc.at[pl.ds(s * ROWS_A + t * WB_CH, WB_CH)])

    @pl.when(s == NS - 1)
    def _():
        pltpu.sync_copy(zb.at[pl.ds(0, TAIL_A)],
                        acc.at[pl.ds(NS * ROWS_A, TAIL_A)])

    plsc.subcore_barrier()
    y_half = y2.at[pl.ds(c * N, N)]

    bufs = (rows_a, rows_b)
    sems = (sem0, sem1)
    cps = [None, None]
    for half in range(2):
        pltpu.sync_copy(srcv.at[s].at[pl.ds(half * CONV_SH, CONV_SH)], src_v)
        pltpu.sync_copy(dstv.at[s].at[pl.ds(half * CONV_SH, CONV_SH)], dst_v)
        cps[0] = pltpu.async_copy(y_half.at[src_v.at[0]], rows_a, sem0)
        for j in range(CONV_SH):
            nxt = j + 1
            if nxt < CONV_SH:
                cps[nxt % 2] = pltpu.async_copy(
                    y_half.at[src_v.at[nxt]], bufs[nxt % 2], sems[nxt % 2])
            cps[j % 2].wait()
            pltpu.sync_copy(bufs[j % 2], acc.at[dst_v.at[j]], add=True)
    plsc.subcore_barrier()
    for t in range(ROWS_A // WB_CH):
        pltpu.sync_copy(acc.at[pl.ds(s * ROWS_A + t * WB_CH, WB_CH)], zb)
        pltpu.sync_copy(
            zb, agg_out.at[pl.ds(c * N + s * ROWS_A + t * WB_CH, WB_CH)])

    @pl.when(s == NS - 1)
    def _():
        pltpu.sync_copy(acc.at[pl.ds(NS * ROWS_A, TAIL_A)],
                        zb.at[pl.ds(0, TAIL_A)])
        pltpu.sync_copy(zb.at[pl.ds(0, TAIL_A)],
                        agg_out.at[pl.ds(c * N + NS * ROWS_A, TAIL_A)])


def _starts_body(b_ref, o_ref):
    i = pl.program_id(0)

    @pl.when(i == 0)
    def _():
        o_ref[...] = jnp.zeros((1, 128), jnp.int32)

    g = lax.broadcasted_iota(jnp.int32, (1, 128), 1)
    cmp = (b_ref[...] < g).astype(jnp.int32)
    o_ref[...] += jnp.sum(cmp, axis=0, keepdims=True)


def _starts_call(batch):
    return pl.pallas_call(
        _starts_body,
        grid=(NBLK,),
        in_specs=[pl.BlockSpec((BLK, 1), lambda i: (i, 0))],
        out_specs=pl.BlockSpec((1, 128), lambda i: (0, 0)),
        out_shape=jax.ShapeDtypeStruct((1, 128), jnp.int32),
    )(batch.reshape(N, 1)).reshape(128)


XCH = 64
TKW = 2048


@functools.partial(
    pl.kernel,
    out_type=[
        jax.ShapeDtypeStruct((N + L,), jnp.float32),
        jax.ShapeDtypeStruct((G, D), jnp.float32),
        jax.ShapeDtypeStruct((G, D), jnp.float32),
        jax.ShapeDtypeStruct((G, L), jnp.float32),
    ],
    mesh=_mesh,
    compiler_params=pltpu.CompilerParams(needs_layout_passes=False),
    scratch_types=[
        pltpu.VMEM((128,), jnp.int32),
        pltpu.VMEM((N,), jnp.uint32),
        pltpu.VMEM((N,), jnp.float32),
        pltpu.VMEM((N + L,), jnp.float32),
        pltpu.VMEM((XCH, D), jnp.float32),
        pltpu.VMEM((2, D), jnp.float32),
        pltpu.VMEM((2, D), jnp.float32),
        pltpu.VMEM((2, L), jnp.float32),
        pltpu.VMEM((L,), jnp.float32),
        pltpu.SemaphoreType.DMA,
    ],
)
def _topk_kernel(starts_hbm, ukey_hbm, keep_hbm, xnew_hbm,
                 keepn_out, mx_out, sm_out, kn_out,
                 starts_v, ukey_v, keep_v, kn_v, xbuf, mx_l, sm_l,
                 knv_l, stg, sem):
    c = lax.axis_index("c")
    s = lax.axis_index("s")
    wid = s * NC + c
    pltpu.sync_copy(starts_hbm, starts_v)

    iota = _iota16()

    starts = [_elem(starts_v, 2 * wid), _elem(starts_v, 2 * wid + 1),
              _elem(starts_v, 2 * wid + 2)]

    base8 = jnp.minimum(lax.div(starts[0], 8) * 8, N - TKW)
    fits = (starts[2] - base8) <= TKW

    @pl.when(fits)
    def _():
        pltpu.sync_copy(ukey_hbm.at[pl.ds(base8, TKW)],
                        ukey_v.at[pl.ds(base8, TKW)])
        pltpu.sync_copy(keep_hbm.at[pl.ds(base8, TKW)],
                        keep_v.at[pl.ds(base8, TKW)])

    @pl.when(jnp.logical_not(fits))
    def _():
        pltpu.sync_copy(ukey_hbm, ukey_v)
        pltpu.sync_copy(keep_hbm, keep_v)

    for gi in range(2):
        ga = starts[gi]
        gb = starts[gi + 1]
        g = 2 * wid + gi
        ca = lax.div(ga, L)
        cb = lax.div(gb + L - 1, L)

        def masked_count(pred_fn):
            def body(jj, acc):
                base = jj * L
                iv = base + iota
                uk = ukey_v[pl.ds(base, L)]
                kf = keep_v[pl.ds(base, L)]
                m = (kf > 0.0) & (iv >= ga) & (iv < gb) & pred_fn(iv, uk)
                return acc + plsc.all_reduce_population_count(m)
            return lax.fori_loop(ca, cb, body, jnp.zeros((L,), jnp.int32))

        cnt = masked_count(lambda iv, uk: jnp.full((L,), True))
        k = lax.shift_right_logical(cnt + 1, 1)

        def bs_step(_, st):
            lo, hi, cgt = st
            mid = lo + lax.shift_right_logical(hi - lo, jnp.uint32(1))
            cm = masked_count(lambda iv, uk, midv=mid: uk > midv)
            ge = cm >= k
            lo2 = jnp.where(ge, mid, lo)
            hi2 = jnp.where(ge, hi, mid)
            cgt2 = jnp.where(ge, cgt, cm)
            return lo2, hi2, cgt2

        ulo = jnp.zeros((L,), jnp.uint32)
        uhi = jnp.full((L,), 0xFFFFFFFF, jnp.uint32)
        _, vthr, cgt = lax.fori_loop(
            0, 32, bs_step, (ulo, uhi, jnp.zeros((L,), jnp.int32)))
        needed = k - cgt

        def ti_step(_, lh):
            lo, hi = lh
            mid = lax.div(lo + hi, 2)
            cm = masked_count(
                lambda iv, uk, midv=mid: (uk == vthr) & (iv <= midv))
            ge = cm >= needed
            return jnp.where(ge, lo, mid + 1), jnp.where(ge, mid, hi)

        ilo = jnp.full((L,), ga - 1, jnp.int32)
        ihi = jnp.full((L,), gb - 1, jnp.int32)
        _, tie_i = lax.fori_loop(0, 14, ti_step, (ilo, ihi))

        kpos = k > 0
        def wr_chunk(jj, carry):
            base = jj * L
            iv = base + iota
            uk = ukey_v[pl.ds(base, L)]
            kf = keep_v[pl.ds(base, L)]
            inb = (iv >= ga) & (iv < gb)
            m = (kf > 0.0) & kpos & ((uk > vthr) | ((uk == vthr) & (iv <= tie_i)))
            vals = jnp.where(m, 1.0, 0.0).astype(jnp.float32)
            kn_v[pl.ds(base, L)] = vals
            tgt = jnp.where(inb, iv, N + iota)
            stg[...] = vals
            pltpu.sync_copy(stg, keepn_out.at[tgt])
            return carry

        lax.fori_loop(ca, cb, wr_chunk, 0)

        for cg in range(D // L):
            mx_l[gi, pl.ds(cg * L, L)] = jnp.full((L,), -jnp.inf, jnp.float32)
            sm_l[gi, pl.ds(cg * L, L)] = jnp.zeros((L,), jnp.float32)

        def ro_chunk(jj, carry):
            base = jj * XCH
            pltpu.async_copy(xnew_hbm.at[pl.ds(base, XCH)], xbuf, sem).wait()

            def ro_row(r, c2):
                i = base + r
                sk = _elem(kn_v, i)
                ok = (sk > 0.0) & (i >= ga) & (i < gb)

                @pl.when(ok)
                def _():
                    for cg in range(D // L):
                        xv = xbuf[r, pl.ds(cg * L, L)]
                        mv = mx_l[gi, pl.ds(cg * L, L)]
                        sv = sm_l[gi, pl.ds(cg * L, L)]
                        mx_l[gi, pl.ds(cg * L, L)] = jnp.maximum(mv, xv)
                        sm_l[gi, pl.ds(cg * L, L)] = sv + xv
                return c2

            lax.fori_loop(0, XCH, ro_row, 0)
            return carry

        xa = lax.div(ga, XCH)
        xb = lax.div(gb + XCH - 1, XCH)
        lax.fori_loop(xa, xb, ro_chunk, 0)
        knv_l[gi] = jnp.where(kpos, k, 0).astype(jnp.float32)

    pltpu.sync_copy(mx_l, mx_out.at[pl.ds(2 * wid, 2)])
    pltpu.sync_copy(sm_l, sm_out.at[pl.ds(2 * wid, 2)])
    pltpu.sync_copy(knv_l, kn_out.at[pl.ds(2 * wid, 2)])


def _mm_body(x_ref, w_ref, ra_ref, rb_ref, kf_ref, y_ref):
    xw = jnp.dot(x_ref[...], w_ref[...], preferred_element_type=jnp.float32)
    rdeg = ra_ref[...] + rb_ref[...]
    dis = kf_ref[...] * lax.rsqrt(rdeg + 1.0)
    y_ref[...] = dis * xw


def _mm_call(x, W, rdeg_part, keepf):
    ra = rdeg_part[:N].reshape(N, 1)
    rb = rdeg_part[N:].reshape(N, 1)
    kf = keepf.reshape(N, 1)
    return pl.pallas_call(
        _mm_body,
        grid=(2, NBLK),
        in_specs=[
            pl.BlockSpec((BLK, D), lambda h, i: (i, 0)),
            pl.BlockSpec((D, DH), lambda h, i: (0, h)),
            pl.BlockSpec((BLK, 1), lambda h, i: (i, 0)),
            pl.BlockSpec((BLK, 1), lambda h, i: (i, 0)),
            pl.BlockSpec((BLK, 1), lambda h, i: (i, 0)),
        ],
        out_specs=pl.BlockSpec((BLK, DH), lambda h, i: (h * NBLK + i, 0)),
        out_shape=jax.ShapeDtypeStruct((NC * N, DH), jnp.float32),
    )(x, W, ra, rb, kf)


def _post_body(a0_ref, a1_ref, y0_ref, y1_ref, ra_ref, rb_ref, kf_ref,
               b_ref, p_ref, xn_ref, uk_ref):
    agg = jnp.concatenate([a0_ref[...], a1_ref[...]], axis=1)
    y = jnp.concatenate([y0_ref[...], y1_ref[...]], axis=1)
    rdeg = ra_ref[...] + rb_ref[...]
    dis = kf_ref[...] * lax.rsqrt(rdeg + 1.0)
    h = jnp.maximum(dis * (agg + y) + b_ref[...], 0.0)
    p = p_ref[...]
    pn = jnp.sqrt(jnp.sum(p * p)) + 1e-16
    sc = jnp.tanh(jnp.dot(h, p.reshape(D, 1),
                          preferred_element_type=jnp.float32) / pn)
    xn_ref[...] = h * sc
    ub = lax.bitcast_convert_type(sc, jnp.uint32)
    uk_ref[...] = jnp.where(ub < jnp.uint32(0x80000000),
                            ub | jnp.uint32(0x80000000),
                            ~ub)


def _post_call(agg2, y2, rdeg_part, keepf, b, p):
    ra = rdeg_part[:N].reshape(N, 1)
    rb = rdeg_part[N:].reshape(N, 1)
    kf = keepf.reshape(N, 1)
    return pl.pallas_call(
        _post_body,
        grid=(NBLK,),
        in_specs=[
            pl.BlockSpec((BLK, DH), lambda i: (i, 0)),
            pl.BlockSpec((BLK, DH), lambda i: (NBLK + i, 0)),
            pl.BlockSpec((BLK, DH), lambda i: (i, 0)),
            pl.BlockSpec((BLK, DH), lambda i: (NBLK + i, 0)),
            pl.BlockSpec((BLK, 1), lambda i: (i, 0)),
            pl.BlockSpec((BLK, 1), lambda i: (i, 0)),
            pl.BlockSpec((BLK, 1), lambda i: (i, 0)),
            pl.BlockSpec((1, D), lambda i: (0, 0)),
            pl.BlockSpec((1, D), lambda i: (0, 0)),
        ],
        out_specs=[
            pl.BlockSpec((BLK, D), lambda i: (i, 0)),
            pl.BlockSpec((BLK, 1), lambda i: (i, 0)),
        ],
        out_shape=[
            jax.ShapeDtypeStruct((N, D), jnp.float32),
            jax.ShapeDtypeStruct((N, 1), jnp.uint32),
        ],
    )(agg2, agg2, y2, y2, ra, rb, kf, b.reshape(1, D), p.reshape(1, D))


def _combine_body(m0, s0, k0, m1, s1, k1, m2, s2, k2, o_ref):
    mx = m0[...] + m1[...] + m2[...]
    mean = (s0[...] / jnp.maximum(k0[...][:, :1], 1.0)
            + s1[...] / jnp.maximum(k1[...][:, :1], 1.0)
            + s2[...] / jnp.maximum(k2[...][:, :1], 1.0))
    o_ref[...] = jnp.concatenate([mx, mean], axis=1)


def _combine_call(parts):
    args = []
    for (mx, sm, kn) in parts:
        args += [mx, sm, kn]
    return pl.pallas_call(
        _combine_body,
        out_shape=jax.ShapeDtypeStruct((G, 2 * D), jnp.float32),
    )(*args)


def kernel(x, edge_index, batch, W0, b0, p0, W1, b1, p1, W2, b2, p2):
    src = edge_index[0]
    dst = edge_index[1]
    src_deg = src.reshape(NC * NS, DEG_E)
    dst_deg = dst.reshape(NC * NS, DEG_E)
    src_conv = src.reshape(NS, CONV_CH, CONV_CW)
    dst_conv = dst.reshape(NS, CONV_CH, CONV_CW)

    keepf = jnp.ones((N,), jnp.float32)
    starts = _starts_call(batch)
    parts = []
    for (W, b, p) in ((W0, b0, p0), (W1, b1, p1), (W2, b2, p2)):
        rdeg_part = _deg_kernel(keepf, src_deg, dst_deg)
        y2 = _mm_call(x, W, rdeg_part, keepf)
        agg2 = _conv_kernel(y2, src_conv, dst_conv)
        x, ukey = _post_call(agg2, y2, rdeg_part, keepf, b, p)
        keep_pad, mx, sm, kn = _topk_kernel(
            starts, ukey.reshape(N), keepf, x)
        keepf = keep_pad[:N]
        parts.append((mx, sm, kn))
    return _combine_call(parts)
